# baseline (speedup 1.0000x reference)
import numpy as np
import jax
import jax.numpy as jnp
from jax.experimental import pallas as pl
from jax.experimental.pallas import tpu as pltpu

D_MODEL = 256
_B = 1024


def _body(i_ref, f_ref, w_ref, out_ref, sum_ref, w2_ref, s_ref):
    i = pl.program_id(0)
    nb = pl.num_programs(0)
    B = out_ref.shape[0]

    @pl.when(i == 0)
    def _init():
        col = jax.lax.broadcasted_iota(jnp.int32, (64, D_MODEL), 1)
        coord = jax.lax.broadcasted_iota(jnp.int32, (64, D_MODEL), 0)
        j = jnp.where(col < 128, col // 2, (col - 128) // 2)
        freq = jnp.exp(j.astype(jnp.float32)
                       * jnp.float32(-2.0 * np.log(10000.0) / 128.0))
        phase = (col % 2).astype(jnp.float32) * jnp.float32(np.pi / 2.0)
        pe = jnp.sin(coord.astype(jnp.float32) * freq + phase)
        pet = jnp.where(col < 128, pe, 0.0)
        peq = jnp.where(col >= 128, pe, 0.0)
        w2_ref[0:128, :] = w_ref[...].astype(jnp.bfloat16)
        w2_ref[128:192, :] = pet.astype(jnp.bfloat16)
        w2_ref[192:256, :] = peq.astype(jnp.bfloat16)
        s_ref[...] = jnp.zeros_like(s_ref)

    col = jax.lax.broadcasted_iota(jnp.int16, (B, D_MODEL), 1)
    g = i_ref[0, 0, 0, :].reshape(B, 1)
    r = i_ref[1, 0, 0, :].reshape(B, 1)
    t = i_ref[2, 0, 0, :].reshape(B, 1)
    q = i_ref[3, 0, 0, :].reshape(B, 1)
    pv = f_ref[0, 0, 0, :].astype(jnp.bfloat16).reshape(B, 1)
    hp = f_ref[1, 0, 0, :].astype(jnp.bfloat16).reshape(B, 1)

    one = jnp.bfloat16(1)
    zero = jnp.bfloat16(0)
    a0 = (jnp.where(col == g, one, zero)
          + jnp.where(col == r, one, zero))
    a1 = (jnp.where(col == t, one, zero)
          + jnp.where(col == q, one, zero))
    a2 = (jnp.where(col == jnp.int16(68), pv, zero)
          + jnp.where(col == jnp.int16(69), hp, zero))
    a3 = jnp.where(col == jnp.int16(70), one, zero)
    mb = (a0 + a1) + (a2 + a3)

    out_ref[...] = jnp.dot(mb, w2_ref[...], preferred_element_type=jnp.float32)

    s_ref[...] += jnp.dot(jnp.ones((1, B), jnp.bfloat16), mb,
                          preferred_element_type=jnp.float32)

    @pl.when(i == nb - 1)
    def _fin():
        sum_ref[...] = jnp.float32(1.0 / (nb * B)) * jnp.dot(
            s_ref[...], w2_ref[...].astype(jnp.float32),
            preferred_element_type=jnp.float32)


def kernel(gate_idx, role_idx, param_val, has_param, grid_positions,
           gate_table, role_table, W_param, b_param):
    N = gate_idx.shape[0]
    nb = N // _B

    w_all = jnp.zeros((128, D_MODEL), jnp.float32)
    w_all = w_all.at[0:64, 0:128].set(gate_table)
    w_all = w_all.at[64:68, 128:192].set(role_table)
    w_all = w_all.at[68, 192:255].set(W_param[0])
    w_all = w_all.at[69, 255].set(1.0)
    w_all = w_all.at[70, 192:255].set(b_param)

    idx16 = jnp.stack([
        gate_idx.astype(jnp.int16),
        role_idx.astype(jnp.int16) + jnp.int16(64),
        grid_positions[:, 0].astype(jnp.int16) + jnp.int16(128),
        grid_positions[:, 1].astype(jnp.int16) + jnp.int16(192),
    ]).reshape(4, nb, 1, _B)
    f2 = jnp.stack([param_val, has_param]).reshape(2, nb, 1, _B)

    out, ssum = pl.pallas_call(
        _body,
        grid=(nb,),
        in_specs=[pl.BlockSpec((4, 1, 1, _B), lambda i: (0, i, 0, 0)),
                  pl.BlockSpec((2, 1, 1, _B), lambda i: (0, i, 0, 0)),
                  pl.BlockSpec((128, D_MODEL), lambda i: (0, 0))],
        out_specs=[pl.BlockSpec((_B, D_MODEL), lambda i: (i, 0)),
                   pl.BlockSpec((1, D_MODEL), lambda i: (0, 0))],
        out_shape=[jax.ShapeDtypeStruct((N, D_MODEL), jnp.float32),
                   jax.ShapeDtypeStruct((1, D_MODEL), jnp.float32)],
        scratch_shapes=[pltpu.VMEM((256, D_MODEL), jnp.bfloat16),
                        pltpu.VMEM((1, D_MODEL), jnp.float32)],
    )(idx16, f2, w_all)

    return out, ssum.reshape(D_MODEL)
